# baseline (speedup 1.0000x reference)
import jax
import jax.numpy as jnp
from jax import lax
from jax.experimental import pallas as pl
from jax.experimental.pallas import tpu as pltpu
from jax.experimental.pallas import tpu_sc as plsc

B = 16
L = 2048
D = 512
PAD_VALUE = 0.0

NUM_CORES = 2
NUM_SUBCORES = 16
NW = NUM_CORES * NUM_SUBCORES
ROWS_W = (B * L) // NW
W_PER_B = L // ROWS_W
CHUNK = 64
ZROWS = 8


def _body(flat_hbm, cu_hbm, out_hbm, cu_v, zbuf, sem):
    wid = lax.axis_index("s") * NUM_CORES + lax.axis_index("c")
    b = wid // W_PER_B
    l0 = (wid % W_PER_B) * ROWS_W

    pltpu.sync_copy(cu_hbm, cu_v)
    v0 = cu_v[pl.ds(0, 16)]
    v1 = cu_v[pl.ds(1, 16)]
    lane = lax.iota(jnp.int32, 16)
    m = lane == b
    zero16 = jnp.zeros((16,), jnp.int32)
    cu_b = jnp.sum(jnp.where(m, v0, zero16))
    seg_len = jnp.sum(jnp.where(m, v1, zero16)) - cu_b
    valid = jnp.clip(seg_len - l0, 0, ROWS_W)

    zv = jnp.zeros((16,), jnp.float32)

    def zero_row(r, carry):
        for cidx in range(D // 16):
            zbuf[r, pl.ds(cidx * 16, 16)] = zv
        return carry

    lax.fori_loop(0, CHUNK, zero_row, 0)

    n_full = valid // CHUNK

    def copy_body(i, carry):
        src = cu_b + l0 + i * CHUNK
        dst = l0 + i * CHUNK
        pltpu.sync_copy(flat_hbm.at[pl.ds(src, CHUNK), :],
                        out_hbm.at[b, pl.ds(dst, CHUNK), :])
        return carry

    lax.fori_loop(0, n_full, copy_body, 0)

    cur = n_full * CHUNK
    rem = valid - cur
    sz = CHUNK // 2
    while sz >= 1:
        do = rem >= sz
        cur_c = cur
        sz_c = sz

        @pl.when(do)
        def _():
            pltpu.sync_copy(flat_hbm.at[pl.ds(cu_b + l0 + cur_c, sz_c), :],
                            out_hbm.at[b, pl.ds(l0 + cur_c, sz_c), :])

        step = jnp.where(do, sz, 0)
        cur = cur + step
        rem = rem - step
        sz //= 2

    n_pad = ROWS_W - valid
    n_pfull = n_pad // CHUNK

    def pad_body(i, carry):
        dst = l0 + valid + i * CHUNK
        pltpu.sync_copy(zbuf, out_hbm.at[b, pl.ds(dst, CHUNK), :])
        return carry

    lax.fori_loop(0, n_pfull, pad_body, 0)

    cur = valid + n_pfull * CHUNK
    rem = ROWS_W - cur
    sz = CHUNK // 2
    while sz >= 1:
        do = rem >= sz
        cur_c = cur
        sz_c = sz

        @pl.when(do)
        def _():
            pltpu.sync_copy(zbuf.at[pl.ds(0, sz_c), :],
                            out_hbm.at[b, pl.ds(l0 + cur_c, sz_c), :])

        step = jnp.where(do, sz, 0)
        cur = cur + step
        rem = rem - step
        sz //= 2


@jax.jit
def kernel(flat, cu_seqlens):
    mesh = plsc.VectorSubcoreMesh(core_axis_name="c", subcore_axis_name="s")
    return pl.kernel(
        _body,
        out_type=jax.ShapeDtypeStruct((B, L, D), jnp.float32),
        mesh=mesh,
        scratch_types=[
            pltpu.VMEM((B + 1,), jnp.int32),
            pltpu.VMEM((CHUNK, D), jnp.float32),
            pltpu.SemaphoreType.DMA,
        ],
        compiler_params=pltpu.CompilerParams(use_tc_tiling_on_sc=False,
                                             needs_layout_passes=False),
    )(flat, cu_seqlens)
